# baseline (speedup 1.0000x reference)
import jax
import jax.numpy as jnp
from jax.experimental import pallas as pl


def _pick_dblk(d: int) -> int:
    best = 8
    for cand in range(8, 1025, 8):
        if d % cand == 0:
            best = cand
    return best


def _body(otu_ref, clr_ref, w_ref, b_ref, out_ref):
    b_idx = pl.program_id(1)
    blk = clr_ref[0]
    nb = blk.shape[1]
    onehot = (jax.lax.broadcasted_iota(jnp.int32, (1, nb), 1) == b_idx)
    col = jnp.sum(blk * onehot.astype(blk.dtype), axis=1, keepdims=True)
    out_ref[0] = otu_ref[...] + col * w_ref[...] + b_ref[...]


def kernel(clr, otu_table, W_val, b_val):
    B, D = clr.shape
    E = otu_table.shape[1]
    dblk = _pick_dblk(D)
    ndb = D // dblk

    clr3 = clr.T.reshape(ndb, dblk, B)
    w_row = W_val[:, 0].reshape(1, E)
    b_row = b_val.reshape(1, E)

    out = pl.pallas_call(
        _body,
        grid=(ndb, B),
        in_specs=[
            pl.BlockSpec((dblk, E), lambda d, b: (d, 0)),
            pl.BlockSpec((1, dblk, B), lambda d, b: (d, 0, 0)),
            pl.BlockSpec((1, E), lambda d, b: (0, 0)),
            pl.BlockSpec((1, E), lambda d, b: (0, 0)),
        ],
        out_specs=pl.BlockSpec((1, dblk, E), lambda d, b: (b, d, 0)),
        out_shape=jax.ShapeDtypeStruct((B, D, E), jnp.float32),
    )(otu_table, clr3, w_row, b_row)
    return out
